# baseline (speedup 1.0000x reference)
import jax
import jax.numpy as jnp
from jax.experimental import pallas as pl
from jax.experimental.pallas import tpu as pltpu

B, SP, ST = 64, 128, 128
E, H = 300, 512
BBLK = 16


def _lstm_act(gates):
    i = gates[:, 0 * H:1 * H]
    g = gates[:, 2 * H:3 * H]
    o = gates[:, 3 * H:4 * H]
    c = jax.nn.sigmoid(i) * jnp.tanh(g)
    return jax.nn.sigmoid(o) * jnp.tanh(c)


def _body(prem_ref, hypot_ref, plen_ref, hlen_ref,
          wihp_ref, bp_ref, wihh_ref, bh_ref,
          wsT_ref, wtT_ref, wmT_ref, waT_ref, whT_ref, bm_ref,
          we_ref, fcwT_ref, fcb_ref,
          out_ref,
          hs_ref, sproj_ref, gh_ref, tproj_ref, pmask_ref, vmask_ref):
    f32 = jnp.float32

    CB = 2
    for cb in range(0, BBLK, CB):
        x = prem_ref[pl.ds(cb, CB)].reshape(CB * SP, E)
        gates = jnp.dot(x, wihp_ref[...], preferred_element_type=f32) + bp_ref[...]
        h = _lstm_act(gates)
        hs_ref[pl.ds(cb, CB)] = h.reshape(CB, SP, H).astype(jnp.bfloat16)
        sp = jnp.dot(h.astype(jnp.bfloat16), wsT_ref[...], preferred_element_type=f32)
        sproj_ref[pl.ds(cb, CB)] = sp.reshape(CB, SP, H).astype(jnp.bfloat16)

    CT = 256 // BBLK
    for ct in range(0, ST, CT):
        x = hypot_ref[pl.ds(ct, CT)].reshape(CT * BBLK, E)
        gates = jnp.dot(x, wihh_ref[...], preferred_element_type=f32) + bh_ref[...]
        h = _lstm_act(gates).astype(jnp.bfloat16)
        gh = jnp.dot(h, whT_ref[...], preferred_element_type=f32) + bm_ref[...]
        gh_ref[pl.ds(ct, CT)] = gh.reshape(CT, BBLK, 4 * H).astype(jnp.bfloat16)
        tp = jnp.dot(h, wtT_ref[...], preferred_element_type=f32)
        tproj_ref[pl.ds(ct, CT)] = tp.reshape(CT, BBLK, H)

    plen = plen_ref[0].reshape(BBLK, 1)
    iota_s = jax.lax.broadcasted_iota(jnp.int32, (BBLK, SP), 1)
    pmask_ref[...] = (iota_s < plen).astype(f32)
    hlen = hlen_ref[0].reshape(1, BBLK, 1)
    iota_t = jax.lax.broadcasted_iota(jnp.int32, (ST, BBLK, 1), 0)
    vmask_ref[...] = (iota_t < hlen).astype(f32)

    def step(k, h_m):
        tpk = tproj_ref[k]
        mproj = jnp.dot(h_m.astype(jnp.bfloat16), wmT_ref[...],
                        preferred_element_type=f32)
        c = (tpk + mproj).astype(jnp.bfloat16)
        HF = BBLK // 2
        a_halves = []
        for lo in (0, HF):
            x3 = jnp.tanh(sproj_ref[pl.ds(lo, HF)] + c[lo:lo + HF][:, None, :])
            e2 = jnp.sum(x3 * we_ref[...], axis=-1).astype(f32)
            e2 = jnp.where(pmask_ref[pl.ds(lo, HF)] > 0.0, e2, -1e9)
            mx = jnp.max(e2, axis=-1, keepdims=True)
            p = jnp.exp(e2 - mx)
            alpha = p / jnp.sum(p, axis=-1, keepdims=True)
            alpha = alpha.astype(jnp.bfloat16)
            a_halves.append(
                jnp.sum(alpha[:, :, None] * hs_ref[pl.ds(lo, HF)], axis=1))
        a = jnp.concatenate(a_halves, axis=0)
        gates = (jnp.dot(a, waT_ref[...], preferred_element_type=f32)
                 + gh_ref[k])
        hnew = _lstm_act(gates)
        vm = vmask_ref[k]
        return jnp.where(vm > 0.0, hnew, h_m)

    h_m = jax.lax.fori_loop(0, ST, step, jnp.zeros((BBLK, H), f32))

    logits = jnp.dot(h_m, fcwT_ref[...], preferred_element_type=f32) + fcb_ref[...]
    mx = jnp.max(logits, axis=-1, keepdims=True)
    sh = logits - mx
    lse = jnp.log(jnp.sum(jnp.exp(sh), axis=-1, keepdims=True))
    out_ref[...] = sh - lse


def _impl(prem_emb, hypo_emb_t, premise_len, hypothesis_len,
          wihpT, bp2, wihhT, bh2, wsT, wtT, wmT, waT, whT, bm2,
          we_bcast, fcwT, fcb2):
    b_loc = prem_emb.shape[0]
    grid = b_loc // BBLK
    plen3 = premise_len.astype(jnp.int32).reshape(grid, BBLK, 1)
    hlen3 = hypothesis_len.astype(jnp.int32).reshape(grid, BBLK, 1)

    full = lambda shape: pl.BlockSpec(shape, lambda i: tuple(0 for _ in shape))
    out = pl.pallas_call(
        _body,
        out_shape=jax.ShapeDtypeStruct((b_loc, 3), jnp.float32),
        grid=(grid,),
        in_specs=[
            pl.BlockSpec((BBLK, SP, E), lambda i: (i, 0, 0)),
            pl.BlockSpec((ST, BBLK, E), lambda i: (0, i, 0)),
            pl.BlockSpec((1, BBLK, 1), lambda i: (i, 0, 0)),
            pl.BlockSpec((1, BBLK, 1), lambda i: (i, 0, 0)),
            full((E, 4 * H)), full((1, 4 * H)),
            full((E, 4 * H)), full((1, 4 * H)),
            full((H, H)), full((H, H)), full((H, H)),
            full((H, 4 * H)), full((H, 4 * H)), full((1, 4 * H)),
            full((1, 1, H)), full((H, 3)), full((1, 3)),
        ],
        out_specs=pl.BlockSpec((BBLK, 3), lambda i: (i, 0)),
        scratch_shapes=[
            pltpu.VMEM((BBLK, SP, H), jnp.bfloat16),
            pltpu.VMEM((BBLK, SP, H), jnp.bfloat16),
            pltpu.VMEM((ST, BBLK, 4 * H), jnp.bfloat16),
            pltpu.VMEM((ST, BBLK, H), jnp.float32),
            pltpu.VMEM((BBLK, SP), jnp.float32),
            pltpu.VMEM((ST, BBLK, 1), jnp.float32),
        ],
        compiler_params=pltpu.CompilerParams(
            dimension_semantics=("parallel",),
            vmem_limit_bytes=56 * 1024 * 1024,
        ),
        name="match_lstm",
    )(prem_emb, hypo_emb_t, plen3, hlen3,
      wihpT, bp2, wihhT, bh2, wsT, wtT, wmT, waT, whT, bm2,
      we_bcast, fcwT, fcb2)
    return out


def kernel(premise, premise_len, hypothesis, hypothesis_len, embed, w_e,
           Ws, Wt, Wm, W_ih_p, b_p, W_ih_h, b_h, W_ih_m, b_m, fc_W, fc_b):
    bf16 = jnp.bfloat16
    prem_emb = embed[premise].astype(bf16)
    hypo_emb_t = jnp.swapaxes(embed[hypothesis], 0, 1).astype(bf16)

    wihpT = W_ih_p.T.astype(bf16)
    wihhT = W_ih_h.T.astype(bf16)
    wsT, wtT, wmT = Ws.T.astype(bf16), Wt.T.astype(bf16), Wm.T.astype(bf16)
    waT = W_ih_m[:, :H].T.astype(bf16)
    whT = W_ih_m[:, H:].T.astype(bf16)
    we_bcast = w_e.reshape(1, 1, H).astype(bf16)
    fcwT = fc_W.T
    bp2 = b_p.reshape(1, 4 * H)
    bh2 = b_h.reshape(1, 4 * H)
    bm2 = b_m.reshape(1, 4 * H)
    fcb2 = fc_b.reshape(1, 3)

    return _impl(prem_emb, hypo_emb_t, premise_len, hypothesis_len,
                 wihpT, bp2, wihhT, bh2, wsT, wtT, wmT, waT, whT, bm2,
                 we_bcast, fcwT, fcb2)


# speedup vs baseline: 1.0852x; 1.0852x over previous
import jax
import jax.numpy as jnp
from jax.experimental import pallas as pl
from jax.experimental.pallas import tpu as pltpu

B, SP, ST = 64, 128, 128
E, H = 300, 512
BBLK = 16


def _lstm_act(gates):
    i = gates[:, 0 * H:1 * H]
    g = gates[:, 2 * H:3 * H]
    o = gates[:, 3 * H:4 * H]
    c = jax.nn.sigmoid(i) * jnp.tanh(g)
    return jax.nn.sigmoid(o) * jnp.tanh(c)


def _body(prem_ref, hypot_ref, plen_ref, hlen_ref,
          wihp_ref, bp_ref, wihh_ref, bh_ref,
          wsT_ref, wtT_ref, wmT_ref, waT_ref, whT_ref, bm_ref,
          we_ref, fcwT_ref, fcb_ref,
          out_ref,
          hs_ref, sproj_ref, gh_ref, tproj_ref, pmask_ref, vmask_ref,
          bdmask_ref):
    f32 = jnp.float32

    CB = 2
    for cb in range(0, BBLK, CB):
        x = prem_ref[pl.ds(cb, CB)].reshape(CB * SP, E)
        gates = jnp.dot(x, wihp_ref[...], preferred_element_type=f32) + bp_ref[...]
        h = _lstm_act(gates)
        hs_ref[pl.ds(cb * SP, CB * SP)] = h.astype(jnp.bfloat16)
        sp = jnp.dot(h.astype(jnp.bfloat16), wsT_ref[...], preferred_element_type=f32)
        sproj_ref[pl.ds(cb, CB)] = sp.reshape(CB, SP, H)

    CT = 256 // BBLK
    for ct in range(0, ST, CT):
        x = hypot_ref[pl.ds(ct, CT)].reshape(CT * BBLK, E)
        gates = jnp.dot(x, wihh_ref[...], preferred_element_type=f32) + bh_ref[...]
        h = _lstm_act(gates).astype(jnp.bfloat16)
        gh = jnp.dot(h, whT_ref[...], preferred_element_type=f32) + bm_ref[...]
        gh_ref[pl.ds(ct, CT)] = gh.reshape(CT, BBLK, 4 * H).astype(jnp.bfloat16)
        tp = jnp.dot(h, wtT_ref[...], preferred_element_type=f32)
        tproj_ref[pl.ds(ct, CT)] = tp.reshape(CT, BBLK, H)

    plen = plen_ref[0].reshape(BBLK, 1)
    iota_s = jax.lax.broadcasted_iota(jnp.int32, (BBLK, SP), 1)
    pmask_ref[...] = (iota_s < plen).astype(f32)
    hlen = hlen_ref[0].reshape(1, BBLK, 1)
    iota_t = jax.lax.broadcasted_iota(jnp.int32, (ST, BBLK, 1), 0)
    vmask_ref[...] = (iota_t < hlen).astype(f32)
    row = jax.lax.broadcasted_iota(jnp.int32, (BBLK, BBLK * SP), 0)
    col = jax.lax.broadcasted_iota(jnp.int32, (BBLK, BBLK * SP), 1)
    bdmask_ref[...] = (col // SP == row).astype(jnp.bfloat16)

    def step(k, h_m):
        tpk = tproj_ref[k]
        mproj = jnp.dot(h_m.astype(jnp.bfloat16), wmT_ref[...],
                        preferred_element_type=f32)
        c = tpk + mproj
        HF = BBLK // 2
        alpha_halves = []
        for lo in (0, HF):
            x3 = jnp.tanh(sproj_ref[pl.ds(lo, HF)] + c[lo:lo + HF][:, None, :])
            e2 = jnp.sum(x3 * we_ref[...], axis=-1)
            e2 = jnp.where(pmask_ref[pl.ds(lo, HF)] > 0.0, e2, -1e9)
            mx = jnp.max(e2, axis=-1, keepdims=True)
            p = jnp.exp(e2 - mx)
            alpha_halves.append(p / jnp.sum(p, axis=-1, keepdims=True))
        alpha = jnp.concatenate(alpha_halves, axis=0).astype(jnp.bfloat16)
        bd = jnp.concatenate([alpha] * BBLK, axis=1) * bdmask_ref[...]
        a = jnp.dot(bd, hs_ref[...], preferred_element_type=f32)
        gates = (jnp.dot(a.astype(jnp.bfloat16), waT_ref[...],
                         preferred_element_type=f32)
                 + gh_ref[k])
        hnew = _lstm_act(gates)
        vm = vmask_ref[k]
        return jnp.where(vm > 0.0, hnew, h_m)

    h_m = jax.lax.fori_loop(0, ST, step, jnp.zeros((BBLK, H), f32))

    logits = jnp.dot(h_m, fcwT_ref[...], preferred_element_type=f32) + fcb_ref[...]
    mx = jnp.max(logits, axis=-1, keepdims=True)
    sh = logits - mx
    lse = jnp.log(jnp.sum(jnp.exp(sh), axis=-1, keepdims=True))
    out_ref[...] = sh - lse


def _impl(prem_emb, hypo_emb_t, premise_len, hypothesis_len,
          wihpT, bp2, wihhT, bh2, wsT, wtT, wmT, waT, whT, bm2,
          we_bcast, fcwT, fcb2):
    b_loc = prem_emb.shape[0]
    grid = b_loc // BBLK
    plen3 = premise_len.astype(jnp.int32).reshape(grid, BBLK, 1)
    hlen3 = hypothesis_len.astype(jnp.int32).reshape(grid, BBLK, 1)

    full = lambda shape: pl.BlockSpec(shape, lambda i: tuple(0 for _ in shape))
    out = pl.pallas_call(
        _body,
        out_shape=jax.ShapeDtypeStruct((b_loc, 3), jnp.float32),
        grid=(grid,),
        in_specs=[
            pl.BlockSpec((BBLK, SP, E), lambda i: (i, 0, 0)),
            pl.BlockSpec((ST, BBLK, E), lambda i: (0, i, 0)),
            pl.BlockSpec((1, BBLK, 1), lambda i: (i, 0, 0)),
            pl.BlockSpec((1, BBLK, 1), lambda i: (i, 0, 0)),
            full((E, 4 * H)), full((1, 4 * H)),
            full((E, 4 * H)), full((1, 4 * H)),
            full((H, H)), full((H, H)), full((H, H)),
            full((H, 4 * H)), full((H, 4 * H)), full((1, 4 * H)),
            full((1, 1, H)), full((H, 3)), full((1, 3)),
        ],
        out_specs=pl.BlockSpec((BBLK, 3), lambda i: (i, 0)),
        scratch_shapes=[
            pltpu.VMEM((BBLK * SP, H), jnp.bfloat16),
            pltpu.VMEM((BBLK, SP, H), jnp.float32),
            pltpu.VMEM((ST, BBLK, 4 * H), jnp.bfloat16),
            pltpu.VMEM((ST, BBLK, H), jnp.float32),
            pltpu.VMEM((BBLK, SP), jnp.float32),
            pltpu.VMEM((ST, BBLK, 1), jnp.float32),
            pltpu.VMEM((BBLK, BBLK * SP), jnp.bfloat16),
        ],
        compiler_params=pltpu.CompilerParams(
            dimension_semantics=("parallel",),
            vmem_limit_bytes=56 * 1024 * 1024,
        ),
        name="match_lstm",
    )(prem_emb, hypo_emb_t, plen3, hlen3,
      wihpT, bp2, wihhT, bh2, wsT, wtT, wmT, waT, whT, bm2,
      we_bcast, fcwT, fcb2)
    return out


def kernel(premise, premise_len, hypothesis, hypothesis_len, embed, w_e,
           Ws, Wt, Wm, W_ih_p, b_p, W_ih_h, b_h, W_ih_m, b_m, fc_W, fc_b):
    bf16 = jnp.bfloat16
    prem_emb = embed[premise].astype(bf16)
    hypo_emb_t = jnp.swapaxes(embed[hypothesis], 0, 1).astype(bf16)

    wihpT = W_ih_p.T.astype(bf16)
    wihhT = W_ih_h.T.astype(bf16)
    wsT, wtT, wmT = Ws.T.astype(bf16), Wt.T.astype(bf16), Wm.T.astype(bf16)
    waT = W_ih_m[:, :H].T.astype(bf16)
    whT = W_ih_m[:, H:].T.astype(bf16)
    we_bcast = w_e.reshape(1, 1, H)
    fcwT = fc_W.T
    bp2 = b_p.reshape(1, 4 * H)
    bh2 = b_h.reshape(1, 4 * H)
    bm2 = b_m.reshape(1, 4 * H)
    fcb2 = fc_b.reshape(1, 3)

    return _impl(prem_emb, hypo_emb_t, premise_len, hypothesis_len,
                 wihpT, bp2, wihhT, bh2, wsT, wtT, wmT, waT, whT, bm2,
                 we_bcast, fcwT, fcb2)
